# Initial kernel scaffold; baseline (speedup 1.0000x reference)
import functools

import jax
import jax.numpy as jnp
from jax import lax
from jax.experimental import pallas as pl
from jax.experimental.pallas import tpu as pltpu
from jax.experimental.pallas import tpu_sc as plsc

D = 64
NT = 7
N_WORKERS = 32
CHUNK = 128


@functools.partial(jax.jit, static_argnums=(1,))
def _sc_embed_call(args, n):
    idx_hbm, tables = args[0], args[1:]
    n_per_w = n // N_WORKERS
    n_chunks = n_per_w // CHUNK
    mesh = plsc.VectorSubcoreMesh(core_axis_name="c", subcore_axis_name="s")

    @functools.partial(
        pl.kernel,
        mesh=mesh,
        out_type=jax.ShapeDtypeStruct((n, D), jnp.float32),
        scratch_types=[
            pltpu.VMEM((NT, CHUNK), jnp.int32),
            pltpu.VMEM((NT, CHUNK, D), jnp.float32),
            pltpu.VMEM((CHUNK, D), jnp.float32),
            pltpu.SemaphoreType.DMA,
        ],
    )
    def sc_embed(idx_ref, t0, t1, t2, t3, t4, t5, t6, out_ref,
                 idx_v, rows_v, out_v, sem):
        tabs = (t0, t1, t2, t3, t4, t5, t6)
        wid = lax.axis_index("s") * 2 + lax.axis_index("c")
        w_base = wid * n_per_w

        def chunk_body(g, carry):
            base = w_base + g * CHUNK
            for t in range(NT):
                pltpu.sync_copy(idx_ref.at[t, pl.ds(base, CHUNK)],
                                idx_v.at[t])
            copies = [
                pltpu.async_copy(tabs[t].at[idx_v.at[t]], rows_v.at[t], sem)
                for t in range(NT)
            ]
            for c in copies:
                c.wait()

            def acc_body(c, carry2):
                for j in range(D // 16):
                    sl = pl.ds(j * 16, 16)
                    v = rows_v[0, c, sl]
                    for t in range(1, NT):
                        v = v + rows_v[t, c, sl]
                    out_v[c, sl] = v
                return carry2

            lax.fori_loop(0, CHUNK, acc_body, 0, unroll=2)
            pltpu.sync_copy(out_v, out_ref.at[pl.ds(base, CHUNK)])
            return carry

        lax.fori_loop(0, n_chunks, chunk_body, 0)

    return sc_embed(idx_hbm, *tables)


def kernel(boxes, input_boxes_counts, w_embed, h_embed, cx_embed, cy_embed,
           xskew_embed, yskew_embed, label_embed):
    del input_boxes_counts
    B, L, _ = boxes.shape
    n = B * L
    idx_t = jnp.transpose(boxes.reshape(n, NT))
    tables = (cx_embed, cy_embed, w_embed, h_embed,
              xskew_embed, yskew_embed, label_embed)
    out = _sc_embed_call((idx_t,) + tables, n)
    return out.reshape(B, L, D)


# speedup vs baseline: 6.0179x; 6.0179x over previous
import functools

import jax
import jax.numpy as jnp
from jax import lax
from jax.experimental import pallas as pl
from jax.experimental.pallas import tpu as pltpu
from jax.experimental.pallas import tpu_sc as plsc

D = 64
NT = 7
N_WORKERS = 32
CHUNK = 128


@functools.partial(jax.jit, static_argnums=(1,))
def _sc_embed_call(args, n):
    idxs, tables = args[:NT], args[NT:]
    n_per_w = n // N_WORKERS
    n_chunks = n_per_w // CHUNK
    mesh = plsc.VectorSubcoreMesh(core_axis_name="c", subcore_axis_name="s")

    @functools.partial(
        pl.kernel,
        mesh=mesh,
        out_type=jax.ShapeDtypeStruct((n, D), jnp.float32),
        scratch_types=[
            pltpu.VMEM((NT, CHUNK), jnp.int32),
            pltpu.VMEM((NT, CHUNK, D), jnp.float32),
            pltpu.VMEM((CHUNK, D), jnp.float32),
            pltpu.SemaphoreType.DMA,
        ],
        compiler_params=pltpu.CompilerParams(use_tc_tiling_on_sc=False),
    )
    def sc_embed(i0, i1, i2, i3, i4, i5, i6, t0, t1, t2, t3, t4, t5, t6,
                 out_ref, idx_v, rows_v, out_v, sem):
        idx_refs = (i0, i1, i2, i3, i4, i5, i6)
        tabs = (t0, t1, t2, t3, t4, t5, t6)
        wid = lax.axis_index("s") * 2 + lax.axis_index("c")
        w_base = wid * n_per_w

        def chunk_body(g, carry):
            base = w_base + g * CHUNK
            for t in range(NT):
                pltpu.sync_copy(idx_refs[t].at[pl.ds(base, CHUNK)],
                                idx_v.at[t])
            copies = [
                pltpu.async_copy(tabs[t].at[idx_v.at[t]], rows_v.at[t], sem)
                for t in range(NT)
            ]
            for c in copies:
                c.wait()

            def acc_body(c, carry2):
                for j in range(D // 16):
                    sl = pl.ds(j * 16, 16)
                    v = rows_v[0, c, sl]
                    for t in range(1, NT):
                        v = v + rows_v[t, c, sl]
                    out_v[c, sl] = v
                return carry2

            lax.fori_loop(0, CHUNK, acc_body, 0, unroll=2)
            pltpu.sync_copy(out_v, out_ref.at[pl.ds(base, CHUNK)])
            return carry

        lax.fori_loop(0, n_chunks, chunk_body, 0)

    return sc_embed(*idxs, *tables)


def kernel(boxes, input_boxes_counts, w_embed, h_embed, cx_embed, cy_embed,
           xskew_embed, yskew_embed, label_embed):
    del input_boxes_counts
    B, L, _ = boxes.shape
    n = B * L
    idx_t = jnp.transpose(boxes.reshape(n, NT))
    idxs = tuple(idx_t[t] for t in range(NT))
    tables = (cx_embed, cy_embed, w_embed, h_embed,
              xskew_embed, yskew_embed, label_embed)
    out = _sc_embed_call(idxs + tables, n)
    return out.reshape(B, L, D)


# speedup vs baseline: 7.6502x; 1.2712x over previous
import functools

import jax
import jax.numpy as jnp
from jax import lax
from jax.experimental import pallas as pl
from jax.experimental.pallas import tpu as pltpu
from jax.experimental.pallas import tpu_sc as plsc

D = 64
NT = 7
N_WORKERS = 32
CHUNK = 64


@functools.partial(jax.jit, static_argnums=(1,))
def _sc_embed_call(args, n):
    idx_blk, tables = args[0], args[1:]
    n_per_w = n // N_WORKERS
    n_chunks = n_per_w // CHUNK
    assert n_chunks % 2 == 0 and n_chunks >= 6
    mesh = plsc.VectorSubcoreMesh(core_axis_name="c", subcore_axis_name="s")

    @functools.partial(
        pl.kernel,
        mesh=mesh,
        out_type=jax.ShapeDtypeStruct((n, D), jnp.float32),
        scratch_types=[
            pltpu.VMEM((2, NT, CHUNK), jnp.int32),
            pltpu.VMEM((2, NT, CHUNK, D), jnp.float32),
            pltpu.VMEM((2, CHUNK, D), jnp.float32),
            pltpu.SemaphoreType.DMA,
            pltpu.SemaphoreType.DMA,
            pltpu.SemaphoreType.DMA,
            pltpu.SemaphoreType.DMA,
            pltpu.SemaphoreType.DMA,
            pltpu.SemaphoreType.DMA,
        ],
        compiler_params=pltpu.CompilerParams(use_tc_tiling_on_sc=False),
    )
    def sc_embed(idx_ref, t0, t1, t2, t3, t4, t5, t6, out_ref,
                 idx_v, rows_v, out_v,
                 sem_g0, sem_g1, sem_i0, sem_i1, sem_o0, sem_o1):
        tabs = (t0, t1, t2, t3, t4, t5, t6)
        sem_g = (sem_g0, sem_g1)
        sem_i = (sem_i0, sem_i1)
        sem_o = (sem_o0, sem_o1)
        wid = lax.axis_index("s") * 2 + lax.axis_index("c")
        w_base = wid * n_chunks

        def issue_idx(ci, p):
            pltpu.async_copy(idx_ref.at[w_base + ci], idx_v.at[p], sem_i[p])

        def wait_idx(p):
            pltpu.make_async_copy(idx_ref.at[0], idx_v.at[p], sem_i[p]).wait()

        def fire_gathers(p):
            for t in range(NT):
                pltpu.async_copy(tabs[t].at[idx_v.at[p, t]],
                                 rows_v.at[p, t], sem_g[p])

        def wait_gathers(p):
            for t in range(NT):
                pltpu.make_async_copy(tabs[t].at[pl.ds(0, CHUNK)],
                                      rows_v.at[p, t], sem_g[p]).wait()

        def accumulate(p):
            def acc_body(c, carry):
                for j in range(D // 16):
                    sl = pl.ds(j * 16, 16)
                    v = rows_v[p, 0, c, sl]
                    for t in range(1, NT):
                        v = v + rows_v[p, t, c, sl]
                    out_v[p, c, sl] = v
                return carry

            lax.fori_loop(0, CHUNK, acc_body, 0, unroll=2)

        def issue_out(ci, p):
            base = (w_base + ci) * CHUNK
            pltpu.async_copy(out_v.at[p], out_ref.at[pl.ds(base, CHUNK)],
                             sem_o[p])

        def wait_out(p):
            pltpu.make_async_copy(out_v.at[p], out_ref.at[pl.ds(0, CHUNK)],
                                  sem_o[p]).wait()

        def step(ci, p, do_next, do_idx2, do_owait):
            if do_next:
                wait_idx(1 - p)
                fire_gathers(1 - p)
            wait_gathers(p)
            if do_idx2:
                issue_idx(ci + 2, p)
            if do_owait:
                wait_out(p)
            accumulate(p)
            issue_out(ci, p)

        issue_idx(0, 0)
        wait_idx(0)
        fire_gathers(0)
        issue_idx(1, 1)

        step(0, 0, True, True, False)
        step(1, 1, True, True, False)

        def pair_body(i, carry):
            ci = 2 * i
            step(ci, 0, True, True, True)
            step(ci + 1, 1, True, True, True)
            return carry

        lax.fori_loop(1, n_chunks // 2 - 1, pair_body, 0)

        step(n_chunks - 2, 0, True, False, True)
        step(n_chunks - 1, 1, False, False, True)
        wait_out(0)
        wait_out(1)

    return sc_embed(idx_blk, *tables)


def kernel(boxes, input_boxes_counts, w_embed, h_embed, cx_embed, cy_embed,
           xskew_embed, yskew_embed, label_embed):
    del input_boxes_counts
    B, L, _ = boxes.shape
    n = B * L
    idx_blk = jnp.transpose(boxes.reshape(n, NT)).reshape(
        NT, n // CHUNK, CHUNK).transpose(1, 0, 2)
    tables = (cx_embed, cy_embed, w_embed, h_embed,
              xskew_embed, yskew_embed, label_embed)
    out = _sc_embed_call((idx_blk,) + tables, n)
    return out.reshape(B, L, D)


# speedup vs baseline: 10.6720x; 1.3950x over previous
import functools

import jax
import jax.numpy as jnp
from jax import lax
from jax.experimental import pallas as pl
from jax.experimental.pallas import tpu as pltpu
from jax.experimental.pallas import tpu_sc as plsc

D = 64
NT = 7
N_WORKERS = 32
CHUNK = 64


@functools.partial(jax.jit, static_argnums=(1,))
def _sc_embed_call(args, n):
    idx_blk, tables = args[0], args[1:]
    n_per_w = n // N_WORKERS
    n_chunks = n_per_w // CHUNK
    assert n_chunks % 2 == 0 and n_chunks >= 6
    mesh = plsc.VectorSubcoreMesh(core_axis_name="c", subcore_axis_name="s")

    @functools.partial(
        pl.kernel,
        mesh=mesh,
        out_type=jax.ShapeDtypeStruct((n, D), jnp.float32),
        scratch_types=[
            pltpu.VMEM((2, NT * CHUNK), jnp.int32),
            pltpu.VMEM((2, NT, CHUNK), jnp.int32),
            pltpu.VMEM((2, NT, CHUNK, D), jnp.float32),
            pltpu.VMEM((2, CHUNK, D), jnp.float32),
            pltpu.SemaphoreType.DMA,
            pltpu.SemaphoreType.DMA,
            pltpu.SemaphoreType.DMA,
            pltpu.SemaphoreType.DMA,
            pltpu.SemaphoreType.DMA,
            pltpu.SemaphoreType.DMA,
        ],
        compiler_params=pltpu.CompilerParams(use_tc_tiling_on_sc=False,
                                             needs_layout_passes=False),
    )
    def sc_embed(idx_ref, t0, t1, t2, t3, t4, t5, t6, out_ref,
                 raw_v, idx_v, rows_v, out_v,
                 sem_g0, sem_g1, sem_i0, sem_i1, sem_o0, sem_o1):
        tabs = (t0, t1, t2, t3, t4, t5, t6)
        sem_g = (sem_g0, sem_g1)
        sem_i = (sem_i0, sem_i1)
        sem_o = (sem_o0, sem_o1)
        wid = lax.axis_index("s") * 2 + lax.axis_index("c")
        w_base = wid * n_chunks
        iota7 = lax.iota(jnp.int32, 16) * NT

        def issue_idx(ci, p):
            base = (w_base + ci) * (NT * CHUNK)
            pltpu.async_copy(idx_ref.at[pl.ds(base, NT * CHUNK)],
                             raw_v.at[p], sem_i[p])

        def wait_idx(p):
            pltpu.make_async_copy(idx_ref.at[pl.ds(0, NT * CHUNK)],
                                  raw_v.at[p], sem_i[p]).wait()

        def transpose_idx(p):
            for t in range(NT):
                for g in range(CHUNK // 16):
                    vals = plsc.load_gather(raw_v.at[p],
                                            [iota7 + (g * 16 * NT + t)])
                    idx_v[p, t, pl.ds(g * 16, 16)] = vals

        def fire_gathers(p):
            for t in range(NT):
                pltpu.async_copy(tabs[t].at[idx_v.at[p, t]],
                                 rows_v.at[p, t], sem_g[p])

        def wait_gathers(p):
            for t in range(NT):
                pltpu.make_async_copy(tabs[t].at[pl.ds(0, CHUNK)],
                                      rows_v.at[p, t], sem_g[p]).wait()

        def accumulate(p):
            def acc_body(c, carry):
                for j in range(D // 16):
                    sl = pl.ds(j * 16, 16)
                    v = rows_v[p, 0, c, sl]
                    for t in range(1, NT):
                        v = v + rows_v[p, t, c, sl]
                    out_v[p, c, sl] = v
                return carry

            lax.fori_loop(0, CHUNK, acc_body, 0, unroll=2)

        def issue_out(ci, p):
            base = (w_base + ci) * CHUNK
            pltpu.async_copy(out_v.at[p], out_ref.at[pl.ds(base, CHUNK)],
                             sem_o[p])

        def wait_out(p):
            pltpu.make_async_copy(out_v.at[p], out_ref.at[pl.ds(0, CHUNK)],
                                  sem_o[p]).wait()

        def step(ci, p, do_next, do_idx2, do_owait):
            if do_next:
                wait_idx(1 - p)
                transpose_idx(1 - p)
                fire_gathers(1 - p)
            wait_gathers(p)
            if do_idx2:
                issue_idx(ci + 2, p)
            if do_owait:
                wait_out(p)
            accumulate(p)
            issue_out(ci, p)

        issue_idx(0, 0)
        wait_idx(0)
        transpose_idx(0)
        fire_gathers(0)
        issue_idx(1, 1)

        step(0, 0, True, True, False)
        step(1, 1, True, True, False)

        def pair_body(i, carry):
            ci = 2 * i
            step(ci, 0, True, True, True)
            step(ci + 1, 1, True, True, True)
            return carry

        lax.fori_loop(1, n_chunks // 2 - 1, pair_body, 0)

        step(n_chunks - 2, 0, True, False, True)
        step(n_chunks - 1, 1, False, False, True)
        wait_out(0)
        wait_out(1)

    return sc_embed(idx_blk, *tables)


def kernel(boxes, input_boxes_counts, w_embed, h_embed, cx_embed, cy_embed,
           xskew_embed, yskew_embed, label_embed):
    del input_boxes_counts
    B, L, _ = boxes.shape
    n = B * L
    idx_flat = boxes.reshape(n * NT)
    tables = (cx_embed, cy_embed, w_embed, h_embed,
              xskew_embed, yskew_embed, label_embed)
    out = _sc_embed_call((idx_flat,) + tables, n)
    return out.reshape(B, L, D)
